# Initial kernel scaffold; baseline (speedup 1.0000x reference)
import jax
import jax.numpy as jnp
from jax.experimental import pallas as pl


def kernel(x, edge_index, edge_weight, W1_rel, b1, W1_root, W2_rel, b2, W2_root):
    raise NotImplementedError("write your pallas kernel here")


# speedup vs baseline: 22.4907x; 22.4907x over previous
import functools

import jax
import jax.numpy as jnp
from jax import lax
from jax.experimental import pallas as pl
from jax.experimental.pallas import tpu as pltpu
from jax.experimental.pallas import tpu_sc as plsc

N = 10000
NP = 10240
E = 640000
NC = 2
NS = 16
CH = 158
PTE = CH * 128
E_PAD = NC * NS * PTE
NPT = NP // NS


def _sc_agg(D):
  mesh = plsc.VectorSubcoreMesh(core_axis_name="c", subcore_axis_name="s",
                                num_cores=NC, num_subcores=NS)
  grp = D // 16

  def body(table, srcr, dstr, wr, out, src_v, dst_v, w_v, r0, r1, bounce,
           acc, sem0, sem1):
    c = lax.axis_index("c")
    s = lax.axis_index("s")
    wid = c * NS + s

    @plsc.parallel_loop(0, NPT, unroll=8)
    def _(i):
      for j in range(grp):
        bounce[i, pl.ds(j * 16, 16)] = jnp.zeros((16,), jnp.float32)

    pltpu.sync_copy(bounce, acc.at[pl.ds(s * NPT, NPT)])

    pltpu.sync_copy(srcr.at[wid], src_v)
    pltpu.sync_copy(dstr.at[wid], dst_v)
    pltpu.sync_copy(wr.at[wid], w_v)
    plsc.subcore_barrier()

    def scale(rows, g):
      @plsc.parallel_loop(0, 8, unroll=2)
      def _(q):
        wv = w_v[g, pl.ds(q * 16, 16)]
        for i in range(16):
          we = wv[i]
          for j in range(grp):
            idx = (q * 16 + i, pl.ds(j * 16, 16))
            rows[idx] = rows[idx] * we

    def gather(g, rbuf, sem):
      pltpu.async_copy(table.at[src_v.at[g]], rbuf, sem)

    def process(g, rbuf, sem):
      pltpu.make_async_copy(table.at[src_v.at[g]], rbuf, sem).wait()
      scale(rbuf, g)
      pltpu.sync_copy(rbuf, acc.at[dst_v.at[g]], add=True)

    gather(0, r0, sem0)

    def step(t, _):
      g0 = 2 * t
      gather(g0 + 1, r1, sem1)
      process(g0, r0, sem0)

      @pl.when(t < CH // 2 - 1)
      def _():
        gather(g0 + 2, r0, sem0)

      process(g0 + 1, r1, sem1)
      return 0

    lax.fori_loop(0, CH // 2, step, 0)
    plsc.subcore_barrier()

    pltpu.sync_copy(acc.at[pl.ds(s * NPT, NPT)], bounce)
    pltpu.sync_copy(bounce, out.at[c, pl.ds(s * NPT, NPT)])

  return pl.kernel(
      body,
      out_type=jax.ShapeDtypeStruct((NC, NP, D), jnp.float32),
      mesh=mesh,
      compiler_params=pltpu.CompilerParams(use_tc_tiling_on_sc=False),
      scratch_types=[
          pltpu.VMEM((CH, 128), jnp.int32),
          pltpu.VMEM((CH, 128), jnp.int32),
          pltpu.VMEM((CH, 128), jnp.float32),
          pltpu.VMEM((128, D), jnp.float32),
          pltpu.VMEM((128, D), jnp.float32),
          pltpu.VMEM((NPT, D), jnp.float32),
          pltpu.VMEM_SHARED((NP, D), jnp.float32),
          pltpu.SemaphoreType.DMA,
          pltpu.SemaphoreType.DMA,
      ],
  )


def _tc1(agg0, agg1, xp, w1rel, b1r, w1root, w2rel, w2root, b2r):
  def body(a0, a1, x_r, wr_r, b1_r, wo_r, w2r_r, w2o_r, b2_r, h2_o, hr_o):
    agg = a0[...] + a1[...]
    h = jnp.dot(agg, wr_r[...], preferred_element_type=jnp.float32)
    h += jnp.dot(x_r[...], wo_r[...], preferred_element_type=jnp.float32)
    h = jnp.maximum(h + b1_r[...], 0.0)
    h2_o[...] = jnp.dot(h, w2r_r[...], preferred_element_type=jnp.float32)
    hr_o[...] = (jnp.dot(h, w2o_r[...], preferred_element_type=jnp.float32)
                 + b2_r[...])

  return pl.pallas_call(
      body,
      out_shape=[
          jax.ShapeDtypeStruct((NP, 16), jnp.float32),
          jax.ShapeDtypeStruct((NP, 8), jnp.float32),
      ],
  )(agg0, agg1, xp, w1rel, b1r, w1root, w2rel, w2root, b2r)


def _tc2(p0, p1, hr):
  def body(p0_r, p1_r, hr_r, o_r):
    o_r[...] = jnp.maximum(p0_r[:, :8] + p1_r[:, :8] + hr_r[...], 0.0)

  return pl.pallas_call(
      body,
      out_shape=jax.ShapeDtypeStruct((NP, 8), jnp.float32),
  )(p0, p1, hr)


def kernel(x, edge_index, edge_weight, W1_rel, b1, W1_root, W2_rel, b2,
           W2_root):
  xp = jnp.pad(x, ((0, NP - N), (0, 32 - x.shape[1])))
  pad = E_PAD - E
  srcr = jnp.pad(edge_index[0], (0, pad)).reshape(NC * NS, CH, 128)
  dstr = jnp.pad(edge_index[1], (0, pad)).reshape(NC * NS, CH, 128)
  wr = jnp.pad(edge_weight, (0, pad)).reshape(NC * NS, CH, 128)

  agg1 = _sc_agg(32)(xp, srcr, dstr, wr)

  w1rel = jnp.pad(W1_rel.T, ((0, 3), (0, 0)))
  w1root = jnp.pad(W1_root.T, ((0, 3), (0, 0)))
  w2rel = jnp.pad(W2_rel.T, ((0, 0), (0, 8)))
  h2p, hr = _tc1(agg1[0], agg1[1], xp, w1rel, b1[None], w1root, w2rel,
                 W2_root.T, b2[None])

  agg2 = _sc_agg(16)(h2p, srcr, dstr, wr)
  return _tc2(agg2[0], agg2[1], hr)[:N]
